# baseline (speedup 1.0000x reference)
import functools

import jax
import jax.numpy as jnp
from jax import lax
from jax.experimental import pallas as pl
from jax.experimental.pallas import tpu as pltpu
from jax.experimental.pallas import tpu_sc as plsc

F = 128
NC = 2
NS = 16
NW = NC * NS


def _make_scatter_stage(acc_rows: int, bt0: int, bt1: int, eb: int):
  rows_per_tile = acc_rows // NS
  assert acc_rows % NS == 0
  btmax = max(bt0, bt1)

  mesh = plsc.VectorSubcoreMesh(core_axis_name="c", subcore_axis_name="s")

  @functools.partial(
      pl.kernel,
      out_type=jax.ShapeDtypeStruct((NC, acc_rows, F), jnp.float32),
      mesh=mesh,
      scratch_types=[
          pltpu.VMEM((btmax, eb), jnp.int32),
          pltpu.VMEM((btmax, eb), jnp.int32),
          pltpu.VMEM((eb, F), jnp.float32),
          pltpu.VMEM_SHARED((acc_rows, F), jnp.float32),
          pltpu.SemaphoreType.DMA,
      ],
  )
  def scat(table_hbm, src_hbm, dst_hbm, zeros_hbm, out_hbm,
           src_v, dst_v, rows_a, acc_sh, sem_a):
    cid = lax.axis_index("c")
    sid = lax.axis_index("s")
    wid = cid * NS + sid
    pltpu.sync_copy(zeros_hbm.at[pl.ds(0, rows_per_tile)],
                    acc_sh.at[pl.ds(sid * rows_per_tile, rows_per_tile)])
    pltpu.sync_copy(src_hbm.at[wid], src_v)
    pltpu.sync_copy(dst_hbm.at[wid], dst_v)
    plsc.subcore_barrier()

    def body(b, carry):
      pltpu.async_copy(table_hbm.at[src_v.at[b]], rows_a, sem_a).wait()
      pltpu.sync_copy(rows_a, acc_sh.at[dst_v.at[b]], add=True)
      return carry

    nb = jnp.where(cid == 0, bt0, bt1)
    lax.fori_loop(0, nb, body, 0)
    plsc.subcore_barrier()
    pltpu.sync_copy(acc_sh.at[pl.ds(sid * rows_per_tile, rows_per_tile)],
                    out_hbm.at[cid, pl.ds(sid * rows_per_tile, rows_per_tile)])

  return scat


def _combine_body(parts_ref, o_ref):
  o_ref[...] = parts_ref[0] + parts_ref[1]


def _final_body(x_ref, v_ref, w_ref, eps_ref, o_ref):
  a = (1.0 + eps_ref[0, 0]) * x_ref[...] + v_ref[0] + v_ref[1]
  o_ref[...] = jnp.dot(a, w_ref[...], preferred_element_type=jnp.float32)


def kernel(X, g1_src, g1_dst, g2_src, g2_dst, W, eps):
  n_nodes, f = X.shape
  n_he = 5000
  e = g1_src.shape[0]
  i32 = jnp.int32

  eb = 128
  w1 = 0.35
  bt1 = -(-int(e * w1 / NS) // eb)
  bt1 += bt1 % 2
  bt0 = -(-(e - NS * bt1 * eb) // (NS * eb))
  bt0 += bt0 % 2
  btmax = max(bt0, bt1)
  c0e = e - NS * bt1 * eb

  def prep(idx, pad_val):
    idx = idx.astype(i32)
    p0 = jnp.concatenate(
        [idx[:c0e], jnp.full((NS * bt0 * eb - c0e,), pad_val, i32)])
    p0 = p0.reshape(NS, bt0, eb)
    p0 = jnp.pad(p0, ((0, 0), (0, btmax - bt0), (0, 0)),
                 constant_values=pad_val)
    pad1 = NS * bt1 * eb - (e - c0e)
    p1 = jnp.concatenate([idx[c0e:], jnp.full((pad1,), pad_val, i32)])
    p1 = p1.reshape(NS, bt1, eb)
    p1 = jnp.pad(p1, ((0, 0), (0, btmax - bt1), (0, 0)),
                 constant_values=pad_val)
    return jnp.concatenate([p0, p1], axis=0)

  s1 = prep(g1_src, n_nodes)
  d1 = prep(g1_dst, 0)
  s2 = prep(g2_src, n_he)
  d2 = prep(g2_dst, 0)

  node_tab_rows = n_nodes + 16
  he_rows = 5120
  node_acc_rows = 10240
  Xt = jnp.zeros((node_tab_rows, f), jnp.float32).at[:n_nodes].set(X)
  zeros = jnp.zeros((node_acc_rows // NS, f), jnp.float32)

  scat1 = _make_scatter_stage(he_rows, bt0, bt1, eb)
  scat2 = _make_scatter_stage(node_acc_rows, bt0, bt1, eb)

  e_parts = scat1(Xt, s1, d1, zeros)

  combine = pl.pallas_call(
      _combine_body,
      out_shape=jax.ShapeDtypeStruct((he_rows, f), jnp.float32),
  )
  Xe = combine(e_parts)

  v_parts = scat2(Xe, s2, d2, zeros)

  rb = 2000
  final = pl.pallas_call(
      _final_body,
      grid=(n_nodes // rb,),
      in_specs=[
          pl.BlockSpec((rb, f), lambda i: (i, 0)),
          pl.BlockSpec((NC, rb, f), lambda i: (0, i, 0)),
          pl.BlockSpec((f, f), lambda i: (0, 0)),
          pl.BlockSpec(memory_space=pltpu.SMEM),
      ],
      out_specs=pl.BlockSpec((rb, f), lambda i: (i, 0)),
      out_shape=jax.ShapeDtypeStruct((n_nodes, f), jnp.float32),
  )
  return final(X, v_parts, W, eps.reshape(1, 1))


# speedup vs baseline: 1.5102x; 1.5102x over previous
import functools

import jax
import jax.numpy as jnp
from jax import lax
from jax.experimental import pallas as pl
from jax.experimental.pallas import tpu as pltpu
from jax.experimental.pallas import tpu_sc as plsc

F = 128
NC = 2
NS = 16
NW = NC * NS


def _make_scatter_stage(acc_rows: int, bt0: int, bt1: int, eb: int):
  rows_per_tile = acc_rows // NS
  assert acc_rows % NS == 0
  btmax = max(bt0, bt1)

  mesh = plsc.VectorSubcoreMesh(core_axis_name="c", subcore_axis_name="s")

  @functools.partial(
      pl.kernel,
      out_type=jax.ShapeDtypeStruct((NC, acc_rows, F), jnp.float32),
      mesh=mesh,
      scratch_types=[
          pltpu.VMEM((btmax, eb), jnp.int32),
          pltpu.VMEM((btmax, eb), jnp.int32),
          pltpu.VMEM((eb, F), jnp.float32),
          pltpu.VMEM_SHARED((acc_rows, F), jnp.float32),
          pltpu.SemaphoreType.DMA,
      ],
  )
  def scat(table_hbm, src_hbm, dst_hbm, zeros_hbm, out_hbm,
           src_v, dst_v, rows_a, acc_sh, sem_a):
    cid = lax.axis_index("c")
    sid = lax.axis_index("s")
    wid = cid * NS + sid
    pltpu.sync_copy(zeros_hbm.at[pl.ds(0, rows_per_tile)],
                    acc_sh.at[pl.ds(sid * rows_per_tile, rows_per_tile)])
    pltpu.sync_copy(src_hbm.at[wid], src_v)
    pltpu.sync_copy(dst_hbm.at[wid], dst_v)
    plsc.subcore_barrier()

    def body(b, carry):
      pltpu.async_copy(table_hbm.at[src_v.at[b]], rows_a, sem_a).wait()
      pltpu.sync_copy(rows_a, acc_sh.at[dst_v.at[b]], add=True)
      return carry

    nb = jnp.where(cid == 0, bt0, bt1)
    lax.fori_loop(0, nb, body, 0)
    plsc.subcore_barrier()
    pltpu.sync_copy(acc_sh.at[pl.ds(sid * rows_per_tile, rows_per_tile)],
                    out_hbm.at[cid, pl.ds(sid * rows_per_tile, rows_per_tile)])

  return scat


def _combine_body(parts_ref, o_ref):
  o_ref[...] = parts_ref[0] + parts_ref[1]


def _final_body(x_ref, v_ref, w_ref, eps_ref, o_ref):
  a = (1.0 + eps_ref[0, 0]) * x_ref[...] + v_ref[0] + v_ref[1]
  o_ref[...] = jnp.dot(a, w_ref[...], preferred_element_type=jnp.float32)


def kernel(X, g1_src, g1_dst, g2_src, g2_dst, W, eps):
  n_nodes, f = X.shape
  n_he = 5000
  e = g1_src.shape[0]
  i32 = jnp.int32

  eb = 128
  bt1 = -(-(e // 2) // (NS * eb))
  bt0 = -(-(e - NS * bt1 * eb) // (NS * eb))
  btmax = max(bt0, bt1)
  c0e = e - NS * bt1 * eb

  def prep(idx, pad_val):
    idx = idx.astype(i32)
    p0 = jnp.concatenate(
        [idx[:c0e], jnp.full((NS * bt0 * eb - c0e,), pad_val, i32)])
    p0 = p0.reshape(NS, bt0, eb)
    p0 = jnp.pad(p0, ((0, 0), (0, btmax - bt0), (0, 0)),
                 constant_values=pad_val)
    pad1 = NS * bt1 * eb - (e - c0e)
    p1 = jnp.concatenate([idx[c0e:], jnp.full((pad1,), pad_val, i32)])
    p1 = p1.reshape(NS, bt1, eb)
    p1 = jnp.pad(p1, ((0, 0), (0, btmax - bt1), (0, 0)),
                 constant_values=pad_val)
    return jnp.concatenate([p0, p1], axis=0)

  s1 = prep(g1_src, n_nodes)
  d1 = prep(g1_dst, 0)
  s2 = prep(g2_src, n_he)
  d2 = prep(g2_dst, 0)

  node_tab_rows = n_nodes + 16
  he_rows = 5120
  node_acc_rows = 10240
  Xt = jnp.zeros((node_tab_rows, f), jnp.float32).at[:n_nodes].set(X)
  zeros = jnp.zeros((node_acc_rows // NS, f), jnp.float32)

  scat1 = _make_scatter_stage(he_rows, bt0, bt1, eb)
  scat2 = _make_scatter_stage(node_acc_rows, bt0, bt1, eb)

  e_parts = scat1(Xt, s1, d1, zeros)

  combine = pl.pallas_call(
      _combine_body,
      out_shape=jax.ShapeDtypeStruct((he_rows, f), jnp.float32),
  )
  Xe = combine(e_parts)

  v_parts = scat2(Xe, s2, d2, zeros)

  rb = 2000
  final = pl.pallas_call(
      _final_body,
      grid=(n_nodes // rb,),
      in_specs=[
          pl.BlockSpec((rb, f), lambda i: (i, 0)),
          pl.BlockSpec((NC, rb, f), lambda i: (0, i, 0)),
          pl.BlockSpec((f, f), lambda i: (0, 0)),
          pl.BlockSpec(memory_space=pltpu.SMEM),
      ],
      out_specs=pl.BlockSpec((rb, f), lambda i: (i, 0)),
      out_shape=jax.ShapeDtypeStruct((n_nodes, f), jnp.float32),
  )
  return final(X, v_parts, W, eps.reshape(1, 1))
